# baseline (speedup 1.0000x reference)
import functools

import jax
import jax.numpy as jnp
from jax import lax
from jax.experimental import pallas as pl
from jax.experimental.pallas import tpu as pltpu, tpu_sc as plsc

BATCH = 16384
EMBED_K = 32

_info = plsc.get_sparse_core_info()
_NC, _NS, _L = _info.num_cores, _info.num_subcores, _info.num_lanes
_NW = _NC * _NS
_BPW = BATCH // _NW
_CHUNK = 128
_NCHUNK = _BPW // _CHUNK
_NGROUP = _BPW // _L


def _sc_body(uidx_hbm, iidx_hbm, user_hbm, item_hbm, w1_hbm, w0_hbm,
             y1_hbm, y0_hbm,
             idx_u, idx_i, urows, irows, w1_v, w0_v, y1_v, y0_v, sem):
    wid = lax.axis_index("s") * _NC + lax.axis_index("c")
    base = wid * _BPW

    pltpu.sync_copy(uidx_hbm.at[wid], idx_u)
    pltpu.sync_copy(iidx_hbm.at[wid], idx_i)
    pltpu.sync_copy(w1_hbm, w1_v)
    pltpu.sync_copy(w0_hbm, w0_v)

    copies = []
    for j in range(_NCHUNK):
        sl = pl.ds(j * _CHUNK, _CHUNK)
        copies.append(pltpu.async_copy(user_hbm.at[idx_u.at[j]], urows.at[sl], sem))
        copies.append(pltpu.async_copy(item_hbm.at[idx_i.at[j]], irows.at[sl], sem))
    for c in copies:
        c.wait()

    iota = lax.broadcasted_iota(jnp.int32, (_L,), 0)

    w1_regs = [w1_v[pl.ds(j * _L, _L)] for j in range(2 * EMBED_K // _L)]
    w0_regs = [w0_v[pl.ds(j * _L, _L)] for j in range(2 * EMBED_K // _L)]

    def _w(regs, k):
        return regs[k // _L][k % _L]

    def group(g, carry):
        rows = g * _L + iota
        acc1 = jnp.zeros((_L,), jnp.float32)
        acc0 = jnp.zeros((_L,), jnp.float32)
        for k in range(EMBED_K):
            col = jnp.full((_L,), k, jnp.int32)
            uv = plsc.load_gather(urows, [rows, col])
            iv = plsc.load_gather(irows, [rows, col])
            acc1 = acc1 + uv * _w(w1_regs, k) + iv * _w(w1_regs, EMBED_K + k)
            acc0 = acc0 + uv * _w(w0_regs, k) + iv * _w(w0_regs, EMBED_K + k)
        y1_v[pl.ds(g * _L, _L)] = acc1
        y0_v[pl.ds(g * _L, _L)] = acc0
        return carry

    lax.fori_loop(0, _NGROUP, group, 0, unroll=False)

    pltpu.sync_copy(y1_v, y1_hbm.at[pl.ds(base, _BPW)])
    pltpu.sync_copy(y0_v, y0_hbm.at[pl.ds(base, _BPW)])


@jax.jit
def _sc_call(uidx, iidx, user_table, item_table, w1, w0):
    mesh = plsc.VectorSubcoreMesh(core_axis_name="c", subcore_axis_name="s")
    f = functools.partial(
        pl.kernel,
        mesh=mesh,
        compiler_params=pltpu.CompilerParams(needs_layout_passes=False,
                                             use_tc_tiling_on_sc=False),
        out_type=(
            jax.ShapeDtypeStruct((BATCH,), jnp.float32),
            jax.ShapeDtypeStruct((BATCH,), jnp.float32),
        ),
        scratch_types=[
            pltpu.VMEM((_NCHUNK, _CHUNK), jnp.int32),
            pltpu.VMEM((_NCHUNK, _CHUNK), jnp.int32),
            pltpu.VMEM((_BPW, EMBED_K), jnp.float32),
            pltpu.VMEM((_BPW, EMBED_K), jnp.float32),
            pltpu.VMEM((2 * EMBED_K,), jnp.float32),
            pltpu.VMEM((2 * EMBED_K,), jnp.float32),
            pltpu.VMEM((_BPW,), jnp.float32),
            pltpu.VMEM((_BPW,), jnp.float32),
            pltpu.SemaphoreType.DMA,
        ],
    )(_sc_body)
    return f(uidx, iidx, user_table, item_table, w1, w0)


def kernel(x, user_table, item_table, W1, W0):
    x = x.astype(jnp.int32)
    uidx = x[:, 0].reshape(_NW, _NCHUNK, _CHUNK)
    iidx = x[:, 1].reshape(_NW, _NCHUNK, _CHUNK)
    w1 = W1.reshape(2 * EMBED_K)
    w0 = W0.reshape(2 * EMBED_K)
    y1, y0 = _sc_call(uidx, iidx, user_table, item_table, w1, w0)
    return (y1.reshape(BATCH, 1), y0.reshape(BATCH, 1))
